# baseline (speedup 1.0000x reference)
import jax
import jax.numpy as jnp
from jax import lax
from jax.experimental import pallas as pl
from jax.experimental.pallas import tpu as pltpu
from jax.experimental.pallas import tpu_sc as plsc

_B = 256
_POOL = 512
_D = 512
_H = 256
_LMAX = 64
_K = 5
_LCHUNK = 8

_LANES = 16
_NSLICES = _POOL // _LANES
_NEGF = -3.0e38
_BIGI = 1 << 30


def _weights_body(q_ref, keys_ref, w1_ref, b1_ref, g1_ref, be1_ref,
                  w2_ref, b2_ref, w_ref):
    q = q_ref[...]
    hpre = q * w1_ref[...] + b1_ref[...]
    mean = jnp.mean(hpre, axis=-1, keepdims=True)
    var = jnp.mean((hpre - mean) ** 2, axis=-1, keepdims=True)
    hn = (hpre - mean) / jnp.sqrt(var + 1e-5) * g1_ref[...] + be1_ref[...]
    hact = jnp.maximum(hn, 0.0)
    query = jnp.dot(hact, w2_ref[...],
                    preferred_element_type=jnp.float32) + b2_ref[...]
    qn = query / jnp.maximum(
        jnp.sqrt(jnp.sum(query * query, axis=-1, keepdims=True)), 1e-8)
    keys = keys_ref[...]
    kn = keys / jnp.maximum(
        jnp.sqrt(jnp.sum(keys * keys, axis=-1, keepdims=True)), 1e-8)
    sims = jax.lax.dot_general(qn, kn, (((1,), (1,)), ((), ())),
                               preferred_element_type=jnp.float32)
    scale = 1.0 + 0.5 * jnp.mean(q)
    s = sims * scale
    m = jnp.max(s, axis=1, keepdims=True)
    e = jnp.exp(s - m)
    w_ref[...] = e / jnp.sum(e, axis=1, keepdims=True)


def _sc_topk_body(w_hbm, ti_hbm, tv_hbm, wbuf, tibuf, tvbuf):
    info = plsc.get_sparse_core_info()
    nc = info.num_cores
    nw = nc * info.num_subcores
    rows_per_w = _B // nw
    wid = lax.axis_index("s") * nc + lax.axis_index("c")
    base = wid * rows_per_w
    lane = lax.iota(jnp.int32, _LANES)
    neg = jnp.full((_LANES,), _NEGF, jnp.float32)
    big = jnp.full((_LANES,), _BIGI, jnp.int32)

    pltpu.sync_copy(w_hbm.at[pl.ds(base, rows_per_w)], wbuf)

    def do_row(r, carry0):

        def scan_slice(j, c):
            v1, v2, v3, v4, v5, i1, i2, i3, i4, i5 = c
            v = wbuf[r, pl.ds(j * _LANES, _LANES)]
            vid = j * _LANES + lane
            g = v > v1
            v1n = jnp.maximum(v1, v)
            i1n = jnp.where(g, vid, i1)
            cv, ci = jnp.minimum(v1, v), jnp.where(g, i1, vid)
            g = cv > v2
            v2n = jnp.maximum(v2, cv)
            i2n = jnp.where(g, ci, i2)
            cv, ci = jnp.minimum(v2, cv), jnp.where(g, i2, ci)
            g = cv > v3
            v3n = jnp.maximum(v3, cv)
            i3n = jnp.where(g, ci, i3)
            cv, ci = jnp.minimum(v3, cv), jnp.where(g, i3, ci)
            g = cv > v4
            v4n = jnp.maximum(v4, cv)
            i4n = jnp.where(g, ci, i4)
            cv, ci = jnp.minimum(v4, cv), jnp.where(g, i4, ci)
            g = cv > v5
            v5n = jnp.maximum(v5, cv)
            i5n = jnp.where(g, ci, i5)
            return (v1n, v2n, v3n, v4n, v5n, i1n, i2n, i3n, i4n, i5n)

        c = lax.fori_loop(0, _NSLICES, scan_slice,
                          (neg, neg, neg, neg, neg, big, big, big, big, big))
        vals = list(c[:5])
        idxs = list(c[5:])

        def allmax(v):
            for s in (8, 4, 2, 1):
                v = jnp.maximum(v, v.at[lane ^ s].get(mode="promise_in_bounds"))
            return v

        def allmin(v):
            for s in (8, 4, 2, 1):
                v = jnp.minimum(v, v.at[lane ^ s].get(mode="promise_in_bounds"))
            return v

        tv = jnp.zeros((_LANES,), jnp.float32)
        ti = big
        for k in range(_K):
            mm = vals[0]
            for t in range(1, _K):
                mm = jnp.maximum(mm, vals[t])
            m = allmax(mm)
            cand = big
            for t in range(_K):
                cand = jnp.minimum(cand, jnp.where(vals[t] == m, idxs[t], big))
            jk = allmin(cand)
            tv = jnp.where(lane == k, m, tv)
            ti = jnp.where(lane == k, jk, ti)
            for t in range(_K):
                vals[t] = jnp.where(idxs[t] == jk, _NEGF, vals[t])

        tibuf[r, pl.ds(0, _LANES)] = ti
        tvbuf[r, pl.ds(0, _LANES)] = tv
        return carry0

    lax.fori_loop(0, rows_per_w, do_row, 0)
    pltpu.sync_copy(tibuf, ti_hbm.at[pl.ds(base, rows_per_w)])
    pltpu.sync_copy(tvbuf, tv_hbm.at[pl.ds(base, rows_per_w)])


def _sc_topk(weights):
    mesh = plsc.VectorSubcoreMesh(core_axis_name="c", subcore_axis_name="s")
    fn = pl.kernel(
        _sc_topk_body,
        out_type=(
            jax.ShapeDtypeStruct((_B, _LANES), jnp.int32),
            jax.ShapeDtypeStruct((_B, _LANES), jnp.float32),
        ),
        mesh=mesh,
        scratch_types=[
            pltpu.VMEM((_B // 32, _POOL), jnp.float32),
            pltpu.VMEM((_B // 32, _LANES), jnp.int32),
            pltpu.VMEM((_B // 32, _LANES), jnp.float32),
        ],
    )
    return fn(weights)


def _combine_body(ti_ref, tv_ref, q_ref, p_ref, o_ref, wm_ref):
    @pl.when(pl.program_id(0) == 0)
    def _expand():
        col = jax.lax.broadcasted_iota(jnp.int32, (_B, _POOL), 1)
        wm = jnp.zeros((_B, _POOL), jnp.float32)
        for k in range(_K):
            wm = jnp.where(col == ti_ref[:, k:k + 1], tv_ref[:, k:k + 1], wm)
        wm_ref[...] = wm

    p = p_ref[...].reshape(_POOL, _LCHUNK * _D)
    acc = jnp.dot(wm_ref[...], p, preferred_element_type=jnp.float32,
                  precision=jax.lax.Precision.HIGHEST)
    q = q_ref[...]
    length = 5.0 + 59.0 * (1.0 - q / 5.0)
    lengths = jnp.clip(jnp.floor(length).astype(jnp.int32), 5, _LMAX)
    i = pl.program_id(0)
    lcol = jax.lax.broadcasted_iota(jnp.int32, (_B, _LCHUNK), 1) + i * _LCHUNK
    lm = (lcol < lengths).astype(jnp.float32)
    o_ref[...] = acc.reshape(_B, _LCHUNK, _D) * lm[:, :, None]


def kernel(x_embed, quality_score, prompt_keys, prompt_embeddings,
           W1, b1, g1, be1, W2, b2):
    del x_embed
    const = lambda i: (0, 0)
    weights = pl.pallas_call(
        _weights_body,
        out_shape=jax.ShapeDtypeStruct((_B, _POOL), jnp.float32),
    )(quality_score, prompt_keys, W1, b1.reshape(1, _H), g1.reshape(1, _H),
      be1.reshape(1, _H), W2, b2.reshape(1, _D))

    topi, topv = _sc_topk(weights)

    prompted = pl.pallas_call(
        _combine_body,
        grid=(_LMAX // _LCHUNK,),
        in_specs=[
            pl.BlockSpec((_B, _LANES), const),
            pl.BlockSpec((_B, _LANES), const),
            pl.BlockSpec((_B, 1), const),
            pl.BlockSpec((_POOL, _LCHUNK, _D), lambda i: (0, i, 0)),
        ],
        out_specs=pl.BlockSpec((_B, _LCHUNK, _D), lambda i: (0, i, 0)),
        out_shape=jax.ShapeDtypeStruct((_B, _LMAX, _D), jnp.float32),
        scratch_shapes=[pltpu.VMEM((_B, _POOL), jnp.float32)],
    )(topi, topv, quality_score, prompt_embeddings)

    return (prompted, jnp.zeros((), jnp.float32))


# speedup vs baseline: 1.5934x; 1.5934x over previous
import jax
import jax.numpy as jnp
from jax import lax
from jax.experimental import pallas as pl
from jax.experimental.pallas import tpu as pltpu
from jax.experimental.pallas import tpu_sc as plsc

_B = 256
_POOL = 512
_D = 512
_H = 256
_LMAX = 64
_K = 5
_LCHUNK = 8

_LANES = 16
_NSLICES = _POOL // _LANES
_NEGF = -3.0e38
_BIGI = 1 << 30


def _weights_body(q_ref, keys_ref, w1_ref, b1_ref, g1_ref, be1_ref,
                  w2_ref, b2_ref, w_ref):
    q = q_ref[...]
    hpre = q * w1_ref[...] + b1_ref[...]
    mean = jnp.mean(hpre, axis=-1, keepdims=True)
    var = jnp.mean((hpre - mean) ** 2, axis=-1, keepdims=True)
    hn = (hpre - mean) / jnp.sqrt(var + 1e-5) * g1_ref[...] + be1_ref[...]
    hact = jnp.maximum(hn, 0.0)
    query = jnp.dot(hact, w2_ref[...],
                    preferred_element_type=jnp.float32) + b2_ref[...]
    qn = query / jnp.maximum(
        jnp.sqrt(jnp.sum(query * query, axis=-1, keepdims=True)), 1e-8)
    keys = keys_ref[...]
    kn = keys / jnp.maximum(
        jnp.sqrt(jnp.sum(keys * keys, axis=-1, keepdims=True)), 1e-8)
    sims = jax.lax.dot_general(qn, kn, (((1,), (1,)), ((), ())),
                               preferred_element_type=jnp.float32)
    scale = 1.0 + 0.5 * jnp.mean(q)
    s = sims * scale
    m = jnp.max(s, axis=1, keepdims=True)
    e = jnp.exp(s - m)
    w_ref[...] = e / jnp.sum(e, axis=1, keepdims=True)


def _sc_topk_body(w_hbm, ti_hbm, tv_hbm, wbuf, tibuf, tvbuf):
    info = plsc.get_sparse_core_info()
    nc = info.num_cores
    nw = nc * info.num_subcores
    rows_per_w = _B // nw
    wid = lax.axis_index("s") * nc + lax.axis_index("c")
    base = wid * rows_per_w
    lane = lax.iota(jnp.int32, _LANES)
    neg = jnp.full((_LANES,), _NEGF, jnp.float32)
    big = jnp.full((_LANES,), _BIGI, jnp.int32)

    pltpu.sync_copy(w_hbm.at[pl.ds(base, rows_per_w)], wbuf)

    def do_row(r, carry0):

        def scan_slice(j, c):
            v1, v2, v3, v4, v5, i1, i2, i3, i4, i5 = c
            v = wbuf[r, pl.ds(j * _LANES, _LANES)]
            vid = j * _LANES + lane
            g = v > v1
            v1n = jnp.maximum(v1, v)
            i1n = jnp.where(g, vid, i1)
            cv, ci = jnp.minimum(v1, v), jnp.where(g, i1, vid)
            g = cv > v2
            v2n = jnp.maximum(v2, cv)
            i2n = jnp.where(g, ci, i2)
            cv, ci = jnp.minimum(v2, cv), jnp.where(g, i2, ci)
            g = cv > v3
            v3n = jnp.maximum(v3, cv)
            i3n = jnp.where(g, ci, i3)
            cv, ci = jnp.minimum(v3, cv), jnp.where(g, i3, ci)
            g = cv > v4
            v4n = jnp.maximum(v4, cv)
            i4n = jnp.where(g, ci, i4)
            cv, ci = jnp.minimum(v4, cv), jnp.where(g, i4, ci)
            g = cv > v5
            v5n = jnp.maximum(v5, cv)
            i5n = jnp.where(g, ci, i5)
            return (v1n, v2n, v3n, v4n, v5n, i1n, i2n, i3n, i4n, i5n)

        c = lax.fori_loop(0, _NSLICES, scan_slice,
                          (neg, neg, neg, neg, neg, big, big, big, big, big))
        vals = list(c[:5])
        idxs = list(c[5:])

        def allmax(v):
            for s in (8, 4, 2, 1):
                v = jnp.maximum(v, v.at[lane ^ s].get(mode="promise_in_bounds"))
            return v

        def allmin(v):
            for s in (8, 4, 2, 1):
                v = jnp.minimum(v, v.at[lane ^ s].get(mode="promise_in_bounds"))
            return v

        tv = jnp.zeros((_LANES,), jnp.float32)
        ti = big
        for k in range(_K):
            mm = vals[0]
            for t in range(1, _K):
                mm = jnp.maximum(mm, vals[t])
            m = allmax(mm)
            cand = big
            for t in range(_K):
                cand = jnp.minimum(cand, jnp.where(vals[t] == m, idxs[t], big))
            jk = allmin(cand)
            tv = jnp.where(lane == k, m, tv)
            ti = jnp.where(lane == k, jk, ti)
            for t in range(_K):
                vals[t] = jnp.where(idxs[t] == jk, _NEGF, vals[t])

        tibuf[r, pl.ds(0, _LANES)] = ti
        tvbuf[r, pl.ds(0, _LANES)] = tv
        return carry0

    lax.fori_loop(0, rows_per_w, do_row, 0)
    pltpu.sync_copy(tibuf, ti_hbm.at[pl.ds(base, rows_per_w)])
    pltpu.sync_copy(tvbuf, tv_hbm.at[pl.ds(base, rows_per_w)])


def _sc_topk(weights):
    mesh = plsc.VectorSubcoreMesh(core_axis_name="c", subcore_axis_name="s")
    fn = pl.kernel(
        _sc_topk_body,
        out_type=(
            jax.ShapeDtypeStruct((_B, _LANES), jnp.int32),
            jax.ShapeDtypeStruct((_B, _LANES), jnp.float32),
        ),
        mesh=mesh,
        scratch_types=[
            pltpu.VMEM((_B // 32, _POOL), jnp.float32),
            pltpu.VMEM((_B // 32, _LANES), jnp.int32),
            pltpu.VMEM((_B // 32, _LANES), jnp.float32),
        ],
    )
    return fn(weights)


def _combine_body(ti_ref, tv_ref, q_ref, p_ref, o_ref, wm_ref):
    @pl.when(pl.program_id(0) == 0)
    def _expand():
        col = jax.lax.broadcasted_iota(jnp.int32, (_B, _POOL), 1)
        wm = jnp.zeros((_B, _POOL), jnp.float32)
        for k in range(_K):
            wm = jnp.where(col == ti_ref[:, k:k + 1], tv_ref[:, k:k + 1], wm)
        wm_ref[...] = wm

    p = p_ref[...].reshape(_POOL, _LCHUNK * _D)
    acc = jnp.dot(wm_ref[...], p, preferred_element_type=jnp.float32)
    q = q_ref[...]
    length = 5.0 + 59.0 * (1.0 - q / 5.0)
    lengths = jnp.clip(jnp.floor(length).astype(jnp.int32), 5, _LMAX)
    i = pl.program_id(0)
    lcol = jax.lax.broadcasted_iota(jnp.int32, (_B, _LCHUNK), 1) + i * _LCHUNK
    lm = (lcol < lengths).astype(jnp.float32)
    o_ref[...] = acc.reshape(_B, _LCHUNK, _D) * lm[:, :, None]


def kernel(x_embed, quality_score, prompt_keys, prompt_embeddings,
           W1, b1, g1, be1, W2, b2):
    del x_embed
    const = lambda i: (0, 0)
    weights = pl.pallas_call(
        _weights_body,
        out_shape=jax.ShapeDtypeStruct((_B, _POOL), jnp.float32),
    )(quality_score, prompt_keys, W1, b1.reshape(1, _H), g1.reshape(1, _H),
      be1.reshape(1, _H), W2, b2.reshape(1, _D))

    topi, topv = _sc_topk(weights)

    prompted = pl.pallas_call(
        _combine_body,
        grid=(_LMAX // _LCHUNK,),
        in_specs=[
            pl.BlockSpec((_B, _LANES), const),
            pl.BlockSpec((_B, _LANES), const),
            pl.BlockSpec((_B, 1), const),
            pl.BlockSpec((_POOL, _LCHUNK, _D), lambda i: (0, i, 0)),
        ],
        out_specs=pl.BlockSpec((_B, _LCHUNK, _D), lambda i: (0, i, 0)),
        out_shape=jax.ShapeDtypeStruct((_B, _LMAX, _D), jnp.float32),
        scratch_shapes=[pltpu.VMEM((_B, _POOL), jnp.float32)],
    )(topi, topv, quality_score, prompt_embeddings)

    return (prompted, jnp.zeros((), jnp.float32))
